# baseline (speedup 1.0000x reference)
import functools

import jax
import jax.numpy as jnp
from jax import lax
from jax.experimental import pallas as pl
from jax.experimental.pallas import tpu as pltpu
from jax.experimental.pallas import tpu_sc as plsc

_B = 16384
_D = 50
_LANES = 16
_NU = 1000000
_NI = 100000
_ROW = 50


@functools.lru_cache(maxsize=None)
def _build(nc: int, ns: int):
    nw = nc * ns
    b_per_w = _B // nw
    groups = b_per_w // _LANES
    mesh = plsc.VectorSubcoreMesh(core_axis_name="c", subcore_axis_name="s")

    @functools.partial(
        pl.kernel,
        mesh=mesh,
        out_type=jax.ShapeDtypeStruct((_B,), jnp.float32),
        compiler_params=pltpu.CompilerParams(
            needs_layout_passes=False,
            use_tc_tiling_on_sc=False,
            disable_bounds_checks=True,
        ),
        scratch_types=[
            pltpu.VMEM((b_per_w,), jnp.int32),
            pltpu.VMEM((b_per_w,), jnp.int32),
            pltpu.VMEM((b_per_w, _ROW), jnp.float32),
            pltpu.VMEM((b_per_w, _ROW), jnp.float32),
            pltpu.VMEM((b_per_w,), jnp.float32),
            pltpu.SemaphoreType.DMA,
            pltpu.SemaphoreType.DMA,
        ],
    )
    def mf_kernel(u_hbm, i_hbm, eu_hbm, ei_hbm, out_hbm,
                  uix, iix, eu_v, ei_v, out_v, sem_u, sem_i):
        wid = lax.axis_index("s") * nc + lax.axis_index("c")
        base = wid * b_per_w
        pltpu.sync_copy(u_hbm.at[pl.ds(base, b_per_w)], uix)
        pltpu.sync_copy(i_hbm.at[pl.ds(base, b_per_w)], iix)
        cp_u = pltpu.async_copy(eu_hbm.at[uix], eu_v, sem_u)
        cp_i = pltpu.async_copy(ei_hbm.at[iix], ei_v, sem_i)
        cp_u.wait()
        cp_i.wait()

        iota = lax.iota(jnp.int32, _LANES)

        def body(g, carry):
            rows = iota + g * _LANES
            acc = jnp.zeros((_LANES,), jnp.float32)
            for d in range(_D):
                cols = jnp.full((_LANES,), d, jnp.int32)
                a = plsc.load_gather(eu_v, [rows, cols])
                b = plsc.load_gather(ei_v, [rows, cols])
                acc = acc + a * b
            out_v[pl.ds(g * _LANES, _LANES)] = acc
            return carry

        lax.fori_loop(0, groups, body, 0)
        pltpu.sync_copy(out_v, out_hbm.at[pl.ds(base, b_per_w)])

    return mf_kernel


def kernel(u, i, Eu, Ei):
    info = plsc.get_sparse_core_info()
    fn = _build(info.num_cores, info.num_subcores)
    return fn(u.astype(jnp.int32), i.astype(jnp.int32), Eu, Ei)


# speedup vs baseline: 4.3202x; 4.3202x over previous
import functools

import jax
import jax.numpy as jnp
from jax import lax
from jax.experimental import pallas as pl
from jax.experimental.pallas import tpu as pltpu
from jax.experimental.pallas import tpu_sc as plsc

_B = 16384
_D = 50
_LANES = 16


@functools.lru_cache(maxsize=None)
def _build(nc: int, ns: int):
    nw = nc * ns
    b_per_w = _B // nw
    half = b_per_w // 2
    mesh = plsc.VectorSubcoreMesh(core_axis_name="c", subcore_axis_name="s")

    @functools.partial(
        pl.kernel,
        mesh=mesh,
        out_type=jax.ShapeDtypeStruct((_B,), jnp.float32),
        compiler_params=pltpu.CompilerParams(
            needs_layout_passes=False,
            use_tc_tiling_on_sc=True,
        ),
        scratch_types=[
            pltpu.VMEM((b_per_w,), jnp.int32),
            pltpu.VMEM((b_per_w,), jnp.int32),
            pltpu.VMEM((half, _D), jnp.float32),
            pltpu.VMEM((half, _D), jnp.float32),
            pltpu.VMEM((b_per_w,), jnp.float32),
            pltpu.SemaphoreType.DMA,
            pltpu.SemaphoreType.DMA,
        ],
    )
    def mf_kernel(u_hbm, i_hbm, eu_hbm, ei_hbm, out_hbm,
                  uvm, ivm, eu_v, ei_v, out_v, sem_u, sem_i):
        wid = lax.axis_index("s") * nc + lax.axis_index("c")
        base = wid * b_per_w
        pltpu.sync_copy(u_hbm.at[pl.ds(base, b_per_w)], uvm)
        pltpu.sync_copy(i_hbm.at[pl.ds(base, b_per_w)], ivm)

        iota = lax.iota(jnp.int32, _LANES)
        tail_mask = iota >= (3 * _LANES - (_D - _LANES))

        for hpass in range(2):
            hbase = hpass * half

            def issue(h, carry):
                uvec = uvm[pl.ds(hbase + h * _LANES, _LANES)]
                ivec = ivm[pl.ds(hbase + h * _LANES, _LANES)]
                for j in range(_LANES):
                    k = h * _LANES + j
                    ru = jnp.sum(jnp.where(iota == j, uvec, 0))
                    ri = jnp.sum(jnp.where(iota == j, ivec, 0))
                    pltpu.async_copy(eu_hbm.at[ru], eu_v.at[k], sem_u)
                    pltpu.async_copy(ei_hbm.at[ri], ei_v.at[k], sem_i)
                return carry

            lax.fori_loop(0, half // _LANES, issue, 0, unroll=False)

            def drain(h, carry):
                for j in range(16):
                    pltpu.make_async_copy(
                        eu_hbm.at[0], eu_v.at[0], sem_u).wait()
                    pltpu.make_async_copy(
                        ei_hbm.at[0], ei_v.at[0], sem_i).wait()
                return carry

            lax.fori_loop(0, half // 16, drain, 0, unroll=False)

            def body(g, carry):
                res = jnp.zeros((_LANES,), jnp.float32)
                for j in range(_LANES):
                    k = g * _LANES + j
                    a0 = eu_v[k, pl.ds(0, _LANES)]
                    a1 = eu_v[k, pl.ds(_LANES, _LANES)]
                    a2 = eu_v[k, pl.ds(2 * _LANES, _LANES)]
                    a3 = eu_v[k, pl.ds(_D - _LANES, _LANES)]
                    c0 = ei_v[k, pl.ds(0, _LANES)]
                    c1 = ei_v[k, pl.ds(_LANES, _LANES)]
                    c2 = ei_v[k, pl.ds(2 * _LANES, _LANES)]
                    c3 = ei_v[k, pl.ds(_D - _LANES, _LANES)]
                    t = a0 * c0 + a1 * c1 + a2 * c2
                    t = t + jnp.where(tail_mask, a3 * c3, 0.0)
                    res = jnp.where(iota == j, jnp.sum(t), res)
                out_v[pl.ds(hbase + g * _LANES, _LANES)] = res
                return carry

            lax.fori_loop(0, half // _LANES, body, 0, unroll=False)

        pltpu.sync_copy(out_v, out_hbm.at[pl.ds(base, b_per_w)])

    return mf_kernel


def kernel(u, i, Eu, Ei):
    info = plsc.get_sparse_core_info()
    fn = _build(info.num_cores, info.num_subcores)
    return fn(u.astype(jnp.int32), i.astype(jnp.int32), Eu, Ei)


# speedup vs baseline: 4.3222x; 1.0005x over previous
import functools

import jax
import jax.numpy as jnp
from jax import lax
from jax.experimental import pallas as pl
from jax.experimental.pallas import tpu as pltpu
from jax.experimental.pallas import tpu_sc as plsc

_B = 16384
_D = 50
_LANES = 16


@functools.lru_cache(maxsize=None)
def _build(nc: int, ns: int):
    nw = nc * ns
    b_per_w = _B // nw
    half = b_per_w // 2
    mesh = plsc.VectorSubcoreMesh(core_axis_name="c", subcore_axis_name="s")

    @functools.partial(
        pl.kernel,
        mesh=mesh,
        out_type=jax.ShapeDtypeStruct((_B,), jnp.float32),
        compiler_params=pltpu.CompilerParams(
            needs_layout_passes=False,
            use_tc_tiling_on_sc=True,
        ),
        scratch_types=[
            pltpu.VMEM((b_per_w,), jnp.int32),
            pltpu.VMEM((b_per_w,), jnp.int32),
            pltpu.VMEM((half, _D), jnp.float32),
            pltpu.VMEM((half, _D), jnp.float32),
            pltpu.VMEM((b_per_w,), jnp.float32),
            pltpu.SemaphoreType.DMA,
            pltpu.SemaphoreType.DMA,
            pltpu.SemaphoreType.DMA,
            pltpu.SemaphoreType.DMA,
            pltpu.SemaphoreType.DMA,
            pltpu.SemaphoreType.DMA,
            pltpu.SemaphoreType.DMA,
            pltpu.SemaphoreType.DMA,
        ],
    )
    def mf_kernel(u_hbm, i_hbm, eu_hbm, ei_hbm, out_hbm,
                  uvm, ivm, eu_v, ei_v, out_v,
                  su0, su1, su2, su3, si0, si1, si2, si3):
        sems_u = (su0, su1, su2, su3)
        sems_i = (si0, si1, si2, si3)
        wid = lax.axis_index("s") * nc + lax.axis_index("c")
        base = wid * b_per_w
        pltpu.sync_copy(u_hbm.at[pl.ds(base, b_per_w)], uvm)
        pltpu.sync_copy(i_hbm.at[pl.ds(base, b_per_w)], ivm)

        iota = lax.iota(jnp.int32, _LANES)
        tail_mask = iota >= (3 * _LANES - (_D - _LANES))

        for hpass in range(2):
            hbase = hpass * half

            def issue(h, carry):
                uvec = uvm[pl.ds(hbase + h * _LANES, _LANES)]
                ivec = ivm[pl.ds(hbase + h * _LANES, _LANES)]
                for j in range(_LANES):
                    k = h * _LANES + j
                    ru = jnp.sum(jnp.where(iota == j, uvec, 0))
                    ri = jnp.sum(jnp.where(iota == j, ivec, 0))
                    pltpu.async_copy(eu_hbm.at[ru], eu_v.at[k], sems_u[j % 4])
                    pltpu.async_copy(ei_hbm.at[ri], ei_v.at[k], sems_i[j % 4])
                return carry

            lax.fori_loop(0, half // _LANES, issue, 0, unroll=False)

            def drain(h, carry):
                for j in range(_LANES):
                    pltpu.make_async_copy(
                        eu_hbm.at[0], eu_v.at[0], sems_u[j % 4]).wait()
                    pltpu.make_async_copy(
                        ei_hbm.at[0], ei_v.at[0], sems_i[j % 4]).wait()
                return carry

            lax.fori_loop(0, half // _LANES, drain, 0, unroll=False)

            def body(g, carry):
                res = jnp.zeros((_LANES,), jnp.float32)
                for j in range(_LANES):
                    k = g * _LANES + j
                    a0 = eu_v[k, pl.ds(0, _LANES)]
                    a1 = eu_v[k, pl.ds(_LANES, _LANES)]
                    a2 = eu_v[k, pl.ds(2 * _LANES, _LANES)]
                    a3 = eu_v[k, pl.ds(_D - _LANES, _LANES)]
                    c0 = ei_v[k, pl.ds(0, _LANES)]
                    c1 = ei_v[k, pl.ds(_LANES, _LANES)]
                    c2 = ei_v[k, pl.ds(2 * _LANES, _LANES)]
                    c3 = ei_v[k, pl.ds(_D - _LANES, _LANES)]
                    t = a0 * c0 + a1 * c1 + a2 * c2
                    t = t + jnp.where(tail_mask, a3 * c3, 0.0)
                    res = jnp.where(iota == j, jnp.sum(t), res)
                out_v[pl.ds(hbase + g * _LANES, _LANES)] = res
                return carry

            lax.fori_loop(0, half // _LANES, body, 0, unroll=False)

        pltpu.sync_copy(out_v, out_hbm.at[pl.ds(base, b_per_w)])

    return mf_kernel


def kernel(u, i, Eu, Ei):
    info = plsc.get_sparse_core_info()
    fn = _build(info.num_cores, info.num_subcores)
    return fn(u.astype(jnp.int32), i.astype(jnp.int32), Eu, Ei)
